# baseline (speedup 1.0000x reference)
import jax
import jax.numpy as jnp
from jax.experimental import pallas as pl

_B, _N, _DIM = 32, 8192, 128
_K = 32
_ROWS = _N // 128

_LOG2E = 1.4426950408889634
_C1 = 0.6931471805599453
_C2 = 0.2402265069591007
_C3 = 0.05550410866482158
_C4 = 0.009618129107628477
_C5 = 0.0013333558146428443


def _fast_exp(t):
    y = t * _LOG2E
    kf = jnp.round(y)
    f = y - kf
    ki = jnp.maximum(kf.astype(jnp.int32), -126)
    scale = jax.lax.bitcast_convert_type((ki + 127) << 23, jnp.float32)
    p = 1.0 + f * (_C1 + f * (_C2 + f * (_C3 + f * (_C4 + f * _C5))))
    return p * scale


def _wta_kernel(x_ref, w_ref, b_ref, out_ref, mask_ref):
    x2d = x_ref[0]
    proj = jax.lax.dot_general(
        x2d, w_ref[...],
        dimension_numbers=(((1,), (1,)), ((), ())),
        preferred_element_type=jnp.float32,
    )
    proj3 = proj.reshape(_ROWS, 128, _DIM) + b_ref[...][None, None, :]
    s = jnp.mean(proj3, axis=-1)

    m0 = jnp.max(s)
    e = _fast_exp(s - m0)
    z = jnp.sum(e)

    ia = jax.lax.broadcasted_iota(jnp.int32, (_ROWS, 128), 0)
    ib = jax.lax.broadcasted_iota(jnp.int32, (_ROWS, 128), 1)
    lin = ia * 128 + ib
    big = jnp.int32(2 ** 30)
    neg = jnp.float32(-jnp.inf)

    sw = s
    msk = jnp.zeros((_ROWS, 128), jnp.float32)
    for _ in range(_K):
        m = jnp.max(sw)
        idx = jnp.min(jnp.where(sw == m, lin, big))
        chosen = lin == idx
        msk = jnp.where(chosen, 1.0, msk)
        sw = jnp.where(chosen, neg, sw)

    w = e * msk * (1.0 / z)
    x3 = x2d.reshape(_ROWS, 128, _DIM)
    out_ref[0, 0, :] = jnp.sum(x3 * w[:, :, None], axis=(0, 1))
    mask_ref[0] = msk


def kernel(x, W, b):
    out, mask3 = pl.pallas_call(
        _wta_kernel,
        grid=(_B,),
        in_specs=[
            pl.BlockSpec((1, _N, _DIM), lambda i: (i, 0, 0)),
            pl.BlockSpec((_DIM, _DIM), lambda i: (0, 0)),
            pl.BlockSpec((_DIM,), lambda i: (0,)),
        ],
        out_specs=[
            pl.BlockSpec((1, 1, _DIM), lambda i: (i, 0, 0)),
            pl.BlockSpec((1, _ROWS, 128), lambda i: (i, 0, 0)),
        ],
        out_shape=[
            jax.ShapeDtypeStruct((_B, 1, _DIM), jnp.float32),
            jax.ShapeDtypeStruct((_B, _ROWS, 128), jnp.float32),
        ],
    )(x, W, b)
    return out.reshape(_B, _DIM), mask3.reshape(_B, _N)


# speedup vs baseline: 4.2971x; 4.2971x over previous
import jax
import jax.numpy as jnp
from jax.experimental import pallas as pl

_B, _N, _DIM = 32, 8192, 128
_K = 32
_ROWS = _N // 128
_BC = 4
_HCH = 4096


def _fast_exp(t):
    u = jnp.maximum(t, -30.0) * (1.0 / 128.0)
    p = 1.0 + u * (1.0 + u * (0.5 + u * (1.0 / 6.0 + u * (
        1.0 / 24.0 + u * (1.0 / 120.0 + u * (1.0 / 720.0))))))
    for _ in range(7):
        p = p * p
    return p


def _wta_kernel(x_ref, w_ref, b_ref, out_ref, mask_ref):
    nh = _N // _HCH
    s_list = []
    for cb in range(_BC):
        s_parts = []
        for h in range(nh):
            proj = jax.lax.dot_general(
                x_ref[cb, h * _HCH:(h + 1) * _HCH, :], w_ref[...],
                dimension_numbers=(((1,), (1,)), ((), ())),
                preferred_element_type=jnp.float32,
            )
            proj3 = (proj.reshape(_HCH // 128, 128, _DIM)
                     + b_ref[...][None, None, :])
            s_parts.append(jnp.mean(proj3, axis=-1))
        s_list.append(jnp.concatenate(s_parts, axis=0))
    s4 = jnp.stack(s_list)

    m0 = jnp.max(s4, axis=(1, 2), keepdims=True)
    e4 = _fast_exp(s4 - m0)
    z4 = jnp.sum(e4, axis=(1, 2), keepdims=True)

    lin = (jax.lax.broadcasted_iota(jnp.int32, (1, _ROWS, 128), 1) * 128
           + jax.lax.broadcasted_iota(jnp.int32, (1, _ROWS, 128), 2))
    big = jnp.int32(2 ** 30)
    neg = jnp.float32(-jnp.inf)

    def body(_, carry):
        sw, msk = carry
        m = jnp.max(sw, axis=(1, 2), keepdims=True)
        idx = jnp.min(jnp.where(sw == m, lin, big), axis=(1, 2), keepdims=True)
        chosen = lin == idx
        msk = jnp.where(chosen, 1.0, msk)
        sw = jnp.where(chosen, neg, sw)
        return sw, msk

    _, msk4 = jax.lax.fori_loop(
        0, _K, body, (s4, jnp.zeros((_BC, _ROWS, 128), jnp.float32)))

    w4 = e4 * msk4 * (1.0 / z4)
    rh = _HCH // 128
    for cb in range(_BC):
        acc = jnp.zeros((_DIM,), jnp.float32)
        for h in range(nh):
            x3 = x_ref[cb, h * _HCH:(h + 1) * _HCH, :].reshape(rh, 128, _DIM)
            wh = w4[cb, h * rh:(h + 1) * rh]
            acc = acc + jnp.sum(x3 * wh[:, :, None], axis=(0, 1))
        out_ref[cb, 0, :] = acc
    mask_ref[...] = msk4


def kernel(x, W, b):
    out, mask3 = pl.pallas_call(
        _wta_kernel,
        grid=(_B // _BC,),
        in_specs=[
            pl.BlockSpec((_BC, _N, _DIM), lambda i: (i, 0, 0)),
            pl.BlockSpec((_DIM, _DIM), lambda i: (0, 0)),
            pl.BlockSpec((_DIM,), lambda i: (0,)),
        ],
        out_specs=[
            pl.BlockSpec((_BC, 1, _DIM), lambda i: (i, 0, 0)),
            pl.BlockSpec((_BC, _ROWS, 128), lambda i: (i, 0, 0)),
        ],
        out_shape=[
            jax.ShapeDtypeStruct((_B, 1, _DIM), jnp.float32),
            jax.ShapeDtypeStruct((_B, _ROWS, 128), jnp.float32),
        ],
    )(x, W, b)
    return out.reshape(_B, _DIM), mask3.reshape(_B, _N)
